# Initial kernel scaffold; baseline (speedup 1.0000x reference)
import jax
import jax.numpy as jnp
from jax.experimental import pallas as pl


def kernel(x, edge_index, batch, W1a, b1a, W1b, b1b, W2a, b2a, W2b, b2b, Wc1, bc1, Wc2, bc2):
    raise NotImplementedError("write your pallas kernel here")


# speedup vs baseline: 2.5503x; 2.5503x over previous
import functools

import jax
import jax.numpy as jnp
from jax import lax
from jax.experimental import pallas as pl
from jax.experimental.pallas import tpu as pltpu
from jax.experimental.pallas import tpu_sc as plsc

NN, NE, DF, HF, NG = 10000, 320000, 128, 128, 64

NWORK = 32
EPW = NE // NWORK
GCH = 200
RPW = 320
NPAD = RPW * NWORK
CAP = 12800
CAPX = CAP + 16
SCH = 3200
VCH = 128

_SC_PARAMS = pltpu.CompilerParams(needs_layout_passes=False)


def _node_transform(x, Wa, ba):
    nb = 5
    blk = NN // nb

    def body(x_ref, w_ref, b_ref, p_ref, q_ref):
        wbot = w_ref[DF:, :]
        wtop = w_ref[:DF, :] - wbot
        xb = x_ref[...]
        p_ref[...] = jnp.dot(xb, wtop, preferred_element_type=jnp.float32) + b_ref[...]
        q_ref[...] = jnp.dot(xb, wbot, preferred_element_type=jnp.float32)

    return pl.pallas_call(
        body,
        grid=(nb,),
        in_specs=[pl.BlockSpec((blk, DF), lambda i: (i, 0)),
                  pl.BlockSpec((2 * DF, HF), lambda i: (0, 0)),
                  pl.BlockSpec((1, HF), lambda i: (0, 0))],
        out_specs=[pl.BlockSpec((blk, HF), lambda i: (i, 0)),
                   pl.BlockSpec((blk, HF), lambda i: (i, 0))],
        out_shape=[jax.ShapeDtypeStruct((NN, HF), jnp.float32),
                   jax.ShapeDtypeStruct((NN, HF), jnp.float32)],
    )(x, Wa, ba.reshape(1, HF))


def _sc_gather_pairsum(P, Q, dst, src):
    mesh = plsc.VectorSubcoreMesh(core_axis_name="c", subcore_axis_name="s")

    @functools.partial(
        pl.kernel,
        out_type=jax.ShapeDtypeStruct((NE, HF), jnp.float32),
        mesh=mesh,
        compiler_params=_SC_PARAMS,
        scratch_types=[pltpu.VMEM((GCH,), jnp.int32),
                       pltpu.VMEM((GCH,), jnp.int32),
                       pltpu.VMEM((GCH,), jnp.int32),
                       pltpu.VMEM((GCH,), jnp.int32),
                       pltpu.VMEM((GCH, HF), jnp.float32),
                       pltpu.VMEM((GCH, HF), jnp.float32),
                       pltpu.SemaphoreType.DMA,
                       pltpu.SemaphoreType.DMA,
                       pltpu.SemaphoreType.DMA,
                       pltpu.SemaphoreType.DMA,
                       pltpu.SemaphoreType.DMA,
                       pltpu.SemaphoreType.DMA],
    )
    def k(p_hbm, q_hbm, dst_hbm, src_hbm, u_hbm,
          idx_d0, idx_s0, idx_d1, idx_s1, rows0, rows1,
          si0, si1, sg0, sg1, sw0, sw1):
        wid = lax.axis_index("s") * 2 + lax.axis_index("c")
        base = wid * EPW
        nc = EPW // GCH
        idxs = ((idx_d0, idx_s0), (idx_d1, idx_s1))
        rows = (rows0, rows1)
        sis = (si0, si1)
        sgs = (sg0, sg1)
        sws = (sw0, sw1)

        def issue_idx(c, b):
            off = base + c * GCH
            pltpu.async_copy(dst_hbm.at[pl.ds(off, GCH)], idxs[b][0], sis[b])
            pltpu.async_copy(src_hbm.at[pl.ds(off, GCH)], idxs[b][1], sis[b])

        issue_idx(0, 0)

        def run_chunk(c, b):
            off = base + c * GCH
            pltpu.make_async_copy(dst_hbm.at[pl.ds(off, GCH)],
                                  idxs[b][0], sis[b]).wait()
            pltpu.make_async_copy(src_hbm.at[pl.ds(off, GCH)],
                                  idxs[b][1], sis[b]).wait()

            @pl.when(c + 1 < nc)
            def _():
                issue_idx(c + 1, 1 - b)

            @pl.when(c >= 2)
            def _():
                pltpu.make_async_copy(
                    rows[b], u_hbm.at[pl.ds(base + (c - 2) * GCH, GCH)],
                    sws[b]).wait()

            pltpu.async_copy(p_hbm.at[idxs[b][0]], rows[b], sgs[b]).wait()
            pltpu.async_copy(q_hbm.at[idxs[b][1]], rows[b], sgs[b],
                             add=True).wait()
            pltpu.async_copy(rows[b], u_hbm.at[pl.ds(off, GCH)], sws[b])

        def pair(p, carry):
            run_chunk(2 * p, 0)
            run_chunk(2 * p + 1, 1)
            return carry

        lax.fori_loop(0, nc // 2, pair, 0)
        pltpu.make_async_copy(
            rows[0], u_hbm.at[pl.ds(base + (nc - 2) * GCH, GCH)], sws[0]).wait()
        pltpu.make_async_copy(
            rows[1], u_hbm.at[pl.ds(base + (nc - 1) * GCH, GCH)], sws[1]).wait()

    return k(P, Q, dst, src)


def _edge_matmul(U, Wb, bb):
    blk = 2000

    def body(u_ref, w_ref, bias_ref, v_ref):
        u = jnp.maximum(u_ref[...], 0.0)
        v_ref[...] = jnp.dot(u, w_ref[...],
                             preferred_element_type=jnp.float32) + bias_ref[...]

    return pl.pallas_call(
        body,
        grid=(NE // blk,),
        in_specs=[pl.BlockSpec((blk, HF), lambda i: (i, 0)),
                  pl.BlockSpec((HF, HF), lambda i: (0, 0)),
                  pl.BlockSpec((1, HF), lambda i: (0, 0))],
        out_specs=pl.BlockSpec((blk, HF), lambda i: (i, 0)),
        out_shape=jax.ShapeDtypeStruct((NE, HF), jnp.float32),
    )(U, Wb, bb.reshape(1, HF))


def _scatter_stage2(v_hbm, h_hbm, ebuf, dbuf, vbufs, acc, vsems, lo, cnt):
    neg = jnp.full((16,), -jnp.inf, dtype=jnp.float32)

    def init_acc(i, carry):
        acc[pl.ds(i * 16, 16)] = neg
        return carry
    lax.fori_loop(0, RPW * HF // 16, init_acc, 0)

    nch = (cnt + VCH - 1) // VCH

    def gather_chunk(kk, buf, sem):
        return pltpu.async_copy(v_hbm.at[ebuf.at[pl.ds(kk * VCH, VCH)]],
                                buf, sem)

    @pl.when(nch > 0)
    def _():
        gather_chunk(0, vbufs[0], vsems[0])

    def consume(kk, vbuf, sem, nbuf, nsem):
        pltpu.make_async_copy(v_hbm.at[ebuf.at[pl.ds(kk * VCH, VCH)]],
                              vbuf, sem).wait()

        @pl.when(kk + 1 < nch)
        def _():
            gather_chunk(kk + 1, nbuf, nsem)

        def upd(g, carry2):
            rvec = (dbuf[pl.ds(kk * VCH + g * 16, 16)] - lo) * HF
            for j in range(16):
                ro = rvec[j]
                vrow = g * 16 + j

                @plsc.parallel_loop(0, HF, 16)
                def _(col):
                    sl = pl.ds(ro + col, 16)
                    acc[sl] = jnp.maximum(acc[sl], vbuf[vrow, pl.ds(col, 16)])
            return carry2

        lax.fori_loop(0, VCH // 16, upd, 0)

    def pair(p, carry):
        @pl.when(2 * p < nch)
        def _():
            consume(2 * p, vbufs[0], vsems[0], vbufs[1], vsems[1])

        @pl.when(2 * p + 1 < nch)
        def _():
            consume(2 * p + 1, vbufs[1], vsems[1], vbufs[0], vsems[0])
        return carry

    lax.fori_loop(0, (nch + 1) // 2, pair, 0)

    def fin(i, carry):
        sl = pl.ds(i * 16, 16)
        v = acc[sl]
        acc[sl] = jnp.where(v == neg, 0.0, v)
        return carry
    lax.fori_loop(0, RPW * HF // 16, fin, 0)

    pltpu.sync_copy(acc.at[pl.ds(0, RPW * HF)],
                    h_hbm.at[pl.ds(lo * HF, RPW * HF)])


def _sc_scatter_max_compact(V, dst):
    mesh = plsc.VectorSubcoreMesh(core_axis_name="c", subcore_axis_name="s")

    @functools.partial(
        pl.kernel,
        out_type=[jax.ShapeDtypeStruct((NPAD * HF,), jnp.float32),
                  jax.ShapeDtypeStruct((NWORK * CAPX,), jnp.int32),
                  jax.ShapeDtypeStruct((NWORK * CAPX,), jnp.int32),
                  jax.ShapeDtypeStruct((NWORK * 16,), jnp.int32)],
        mesh=mesh,
        compiler_params=_SC_PARAMS,
        scratch_types=[pltpu.VMEM((SCH,), jnp.int32),
                       pltpu.VMEM((SCH,), jnp.int32),
                       pltpu.VMEM((CAPX,), jnp.int32),
                       pltpu.VMEM((CAPX,), jnp.int32),
                       pltpu.VMEM((VCH, HF), jnp.float32),
                       pltpu.VMEM((VCH, HF), jnp.float32),
                       pltpu.VMEM(((RPW + 8) * HF,), jnp.float32),
                       pltpu.VMEM((16,), jnp.int32),
                       pltpu.SemaphoreType.DMA,
                       pltpu.SemaphoreType.DMA,
                       pltpu.SemaphoreType.DMA,
                       pltpu.SemaphoreType.DMA],
    )
    def k(v_hbm, dst_hbm, h_hbm, e_hbm, d_hbm, c_hbm,
          dchunk0, dchunk1, ebuf, dbuf, vbuf0, vbuf1, acc, cvec,
          dsem0, dsem1, vsem0, vsem1):
        wid = lax.axis_index("s") * 2 + lax.axis_index("c")
        lo = wid * RPW
        hi = lo + RPW
        lane = lax.iota(jnp.int32, 16)
        zero16 = jnp.zeros((16,), dtype=jnp.int32)

        dummy = jnp.broadcast_to(lo + RPW, (16,)).astype(jnp.int32)

        def init_ebuf(i, carry):
            ebuf[pl.ds(i * 16, 16)] = zero16
            dbuf[pl.ds(i * 16, 16)] = dummy
            return carry
        lax.fori_loop(0, CAPX // 16, init_ebuf, 0)

        dnums = lax.GatherDimensionNumbers(
            offset_dims=(), collapsed_slice_dims=(0,), start_index_map=(0,))

        dchunks = (dchunk0, dchunk1)
        dsems = (dsem0, dsem1)
        nchs = NE // SCH
        pltpu.async_copy(dst_hbm.at[pl.ds(0, SCH)], dchunk0, dsem0)

        def scan_chunk(c, cnt, dchunk):

            def grp(i, cnt):
                d16 = dchunk[pl.ds(i * 16, 16)]
                m = (d16 >= lo) & (d16 < hi)
                s = jnp.where(m, 1, 0)
                for st in (1, 2, 4, 8):
                    sh = lax.gather(
                        s, jnp.maximum(lane - st, 0)[:, None], dnums, (1,),
                        mode=lax.GatherScatterMode.PROMISE_IN_BOUNDS)
                    s = s + jnp.where(lane >= st, sh, 0)
                pos = jnp.where(m, cnt + s - 1, CAP + lane)
                eid = c * SCH + i * 16 + lane
                plsc.store_scatter(ebuf, [pos], eid)
                plsc.store_scatter(dbuf, [pos], d16)
                return cnt + s[15]

            return plsc.parallel_loop(0, SCH // 16, carry=cnt)(grp)

        def scan_pair(p, cnt):
            for b in (0, 1):
                c = 2 * p + b
                pltpu.make_async_copy(dst_hbm.at[pl.ds(c * SCH, SCH)],
                                      dchunks[b], dsems[b]).wait()

                @pl.when(c + 1 < nchs)
                def _():
                    pltpu.async_copy(
                        dst_hbm.at[pl.ds((c + 1) * SCH, SCH)],
                        dchunks[1 - b], dsems[1 - b])

                cnt = scan_chunk(c, cnt, dchunks[b])
            return cnt

        cnt = lax.fori_loop(0, nchs // 2, scan_pair, jnp.int32(0))

        pltpu.sync_copy(ebuf, e_hbm.at[pl.ds(wid * CAPX, CAPX)])
        pltpu.sync_copy(dbuf, d_hbm.at[pl.ds(wid * CAPX, CAPX)])
        cvec[pl.ds(0, 16)] = jnp.broadcast_to(cnt, (16,)).astype(jnp.int32)
        pltpu.sync_copy(cvec, c_hbm.at[pl.ds(wid * 16, 16)])

        _scatter_stage2(v_hbm, h_hbm, ebuf, dbuf, (vbuf0, vbuf1), acc,
                        (vsem0, vsem1), lo, cnt)

    return k(V, dst)


def _sc_scatter_max_reuse(V, elists, dlists, cnts):
    mesh = plsc.VectorSubcoreMesh(core_axis_name="c", subcore_axis_name="s")

    @functools.partial(
        pl.kernel,
        out_type=jax.ShapeDtypeStruct((NPAD * HF,), jnp.float32),
        mesh=mesh,
        compiler_params=_SC_PARAMS,
        scratch_types=[pltpu.VMEM((CAPX,), jnp.int32),
                       pltpu.VMEM((CAPX,), jnp.int32),
                       pltpu.VMEM((VCH, HF), jnp.float32),
                       pltpu.VMEM((VCH, HF), jnp.float32),
                       pltpu.VMEM(((RPW + 8) * HF,), jnp.float32),
                       pltpu.VMEM((16,), jnp.int32),
                       pltpu.SemaphoreType.DMA,
                       pltpu.SemaphoreType.DMA],
    )
    def k(v_hbm, e_hbm, d_hbm, c_hbm, h_hbm,
          ebuf, dbuf, vbuf0, vbuf1, acc, cvec, vsem0, vsem1):
        wid = lax.axis_index("s") * 2 + lax.axis_index("c")
        lo = wid * RPW
        pltpu.sync_copy(e_hbm.at[pl.ds(wid * CAPX, CAPX)], ebuf)
        pltpu.sync_copy(d_hbm.at[pl.ds(wid * CAPX, CAPX)], dbuf)
        pltpu.sync_copy(c_hbm.at[pl.ds(wid * 16, 16)], cvec)
        cnt = cvec[pl.ds(0, 16)][0]
        _scatter_stage2(v_hbm, h_hbm, ebuf, dbuf, (vbuf0, vbuf1), acc,
                        (vsem0, vsem1), lo, cnt)

    return k(V, elists, dlists, cnts)


def _pool_classify(h, batch2d, Wc1, bc1, Wc2, bc2):
    def body(h_ref, b_ref, w1_ref, b1_ref, w2_ref, b2_ref, o_ref):
        gids = lax.broadcasted_iota(jnp.int32, (NG, NN), 0)
        onehot = (b_ref[...] == gids).astype(jnp.float32)
        h32 = h_ref[...]
        sums = jnp.dot(onehot, h32, preferred_element_type=jnp.float32)
        counts = jnp.sum(onehot, axis=1, keepdims=True)
        pooled = sums / jnp.maximum(counts, 1.0)
        z = jnp.maximum(
            jnp.dot(pooled, w1_ref[...], preferred_element_type=jnp.float32)
            + b1_ref[...], 0.0)
        o_ref[...] = jnp.dot(z, w2_ref[...],
                             preferred_element_type=jnp.float32) + b2_ref[...]

    return pl.pallas_call(
        body,
        in_specs=[pl.BlockSpec((NN, HF), lambda: (0, 0)),
                  pl.BlockSpec((1, NN), lambda: (0, 0)),
                  pl.BlockSpec((HF, 64), lambda: (0, 0)),
                  pl.BlockSpec((1, 64), lambda: (0, 0)),
                  pl.BlockSpec((64, 2), lambda: (0, 0)),
                  pl.BlockSpec((1, 2), lambda: (0, 0))],
        out_specs=pl.BlockSpec((NG, 2), lambda: (0, 0)),
        out_shape=jax.ShapeDtypeStruct((NG, 2), jnp.float32),
    )(h, batch2d, Wc1, bc1.reshape(1, 64), Wc2, bc2.reshape(1, 2))


def kernel(x, edge_index, batch, W1a, b1a, W1b, b1b, W2a, b2a, W2b, b2b,
           Wc1, bc1, Wc2, bc2):
    src = edge_index[0]
    dst = edge_index[1]

    P1, Q1 = _node_transform(x, W1a, b1a)
    U1 = _sc_gather_pairsum(P1, Q1, dst, src)
    V1 = _edge_matmul(U1, W1b, b1b)
    h1f, elists, dlists, cnts = _sc_scatter_max_compact(V1, dst)
    h1 = h1f.reshape(NPAD, HF)

    P2, Q2 = _node_transform(h1[:NN], W2a, b2a)
    U2 = _sc_gather_pairsum(P2, Q2, dst, src)
    V2 = _edge_matmul(U2, W2b, b2b)
    h2 = _sc_scatter_max_reuse(V2, elists, dlists, cnts).reshape(NPAD, HF)

    return _pool_classify(h2[:NN], batch.reshape(1, NN), Wc1, bc1, Wc2, bc2)
